# baseline (speedup 1.0000x reference)
import functools
import jax
import jax.numpy as jnp
from jax import lax
from jax.experimental import pallas as pl
from jax.experimental.pallas import tpu as pltpu, tpu_sc as plsc

W = 20
B = 4096
DIM = 64
L = 16

_NC, _NS = 2, 16
NW = _NC * _NS
BPW = B // NW
GROUPS = BPW // L
DCH = DIM // L

N_ITEMS = 1000000
N_PAIRS = N_ITEMS // 2


def _splat_lane(v, j):
    idx = jnp.full((L, 1), j, dtype=jnp.int32)
    dnums = lax.GatherDimensionNumbers(
        offset_dims=(), collapsed_slice_dims=(0,), start_index_map=(0,))
    return lax.gather(v, idx, dnums, slice_sizes=(1,),
                      mode=lax.GatherScatterMode.PROMISE_IN_BOUNDS)


def _build(interpret=False):
    mesh = plsc.VectorSubcoreMesh(
        core_axis_name="c", subcore_axis_name="s",
        num_cores=_NC, num_subcores=_NS)

    @functools.partial(
        pl.kernel,
        out_type=jax.ShapeDtypeStruct((B, DIM), jnp.float32),
        mesh=mesh,
        scratch_types=[
            pltpu.VMEM((W, BPW), jnp.int32),
            pltpu.VMEM((W, BPW), jnp.float32),
            pltpu.VMEM((W, BPW), jnp.float32),
            pltpu.VMEM((W * BPW,), jnp.float32),
            pltpu.VMEM((W * BPW,), jnp.float32),
            pltpu.VMEM((W * BPW,), jnp.float32),
            pltpu.VMEM((BPW,), jnp.float32),
            pltpu.VMEM((1, 2 * DIM), jnp.float32),
            pltpu.VMEM((BPW, 2 * DIM), jnp.float32),
            pltpu.VMEM((BPW, 2 * DIM), jnp.float32),
            pltpu.VMEM((BPW, DIM), jnp.float32),
            pltpu.SemaphoreType.DMA,
            pltpu.SemaphoreType.DMA,
        ],
        compiler_params=pltpu.CompilerParams(use_tc_tiling_on_sc=True),
        interpret=interpret,
    )
    def sc_kernel(pairs_hbm, last_hbm, idx_hbm, rew_hbm, live_hbm, out_hbm,
                  pidx_v, rew_v, live_v, wts_v, bit_v, msk_v, cpad_v, last_v,
                  rows0, rows1, acc_v, sem0, sem1):
        wid = lax.axis_index("s") * _NC + lax.axis_index("c")
        base = wid * BPW

        pltpu.sync_copy(idx_hbm.at[:, pl.ds(base, BPW)], pidx_v)
        pltpu.sync_copy(rew_hbm.at[:, pl.ds(base, BPW)], rew_v)
        pltpu.sync_copy(live_hbm.at[:, pl.ds(base, BPW)], live_v)
        pltpu.sync_copy(last_hbm, last_v)

        def pre_body(w, _):
            for c in range(GROUPS):
                sl = pl.ds(c * L, L)
                fsl = pl.ds(w * BPW + c * L, L)
                iv = pidx_v[w, sl]
                bit_v[fsl] = jnp.bitwise_and(iv, 1).astype(jnp.float32)
                df = (iv - N_ITEMS).astype(jnp.float32)
                msk_v[fsl] = jnp.maximum(1.0 - df * df, 0.0)
                pidx_v[w, sl] = jnp.minimum(
                    jnp.right_shift(iv, 1), N_PAIRS - 1)
            return 0

        lax.fori_loop(0, W, pre_body, 0)

        rows = (rows0, rows1)
        sems = (sem0, sem1)
        pltpu.make_async_copy(pairs_hbm.at[pidx_v.at[0]], rows0, sem0).start()

        def wts_body(c, _):
            sl = pl.ds(c * L, L)
            cnt = live_v[0, sl]
            for w in range(1, W):
                cnt = cnt + live_v[w, sl]
            rcp = 1.0 / cnt
            cp = jnp.zeros((L,), jnp.float32)
            for w in range(W):
                fsl = pl.ds(w * BPW + c * L, L)
                wt = rew_v[w, sl] * live_v[w, sl] * rcp
                pm = msk_v[fsl]
                wts_v[fsl] = wt * (1.0 - pm)
                cp = cp + wt * pm
            cpad_v[sl] = cp
            return 0

        lax.fori_loop(0, GROUPS, wts_body, 0)

        def process(w, rbuf, first):
            def group_body(g, _):
                w16 = wts_v[pl.ds(w * BPW + g * L, L)]
                b16 = bit_v[pl.ds(w * BPW + g * L, L)]
                for j in range(L):
                    wv = _splat_lane(w16, j)
                    bv = _splat_lane(b16, j)
                    r = g * L + j
                    for d in range(DCH):
                        sl = pl.ds(d * L, L)
                        lo = rbuf[r, sl]
                        hi = rbuf[r, pl.ds(DIM + d * L, L)]
                        prod = (lo + (hi - lo) * bv) * wv
                        if first:
                            acc_v[r, sl] = prod
                        else:
                            plsc.addupdate(acc_v.at[r, sl], prod)
                return 0

            lax.fori_loop(0, GROUPS, group_body, 0)

        def start(w, buf_i):
            pltpu.make_async_copy(
                pairs_hbm.at[pidx_v.at[w]], rows[buf_i], sems[buf_i]).start()

        def wait(buf_i):
            pltpu.make_async_copy(
                pairs_hbm.at[pidx_v.at[0]], rows[buf_i], sems[buf_i]).wait()

        wait(0)
        start(1, 1)
        process(0, rows0, True)
        wait(1)
        start(2, 0)
        process(1, rows1, False)

        def pair_body(i, _):
            we = 2 + 2 * i
            wait(0)
            start(we + 1, 1)
            process(we, rows0, False)
            wait(1)

            @pl.when(we + 2 < W)
            def _():
                start(we + 2, 0)

            process(we + 1, rows1, False)
            return 0

        lax.fori_loop(0, (W - 2) // 2, pair_body, 0)

        def fix_body(g, _):
            c16 = cpad_v[pl.ds(g * L, L)]
            for j in range(L):
                cv = _splat_lane(c16, j)
                r = g * L + j
                for d in range(DCH):
                    sl = pl.ds(d * L, L)
                    plsc.addupdate(acc_v.at[r, sl], last_v[0, sl] * cv)
            return 0

        lax.fori_loop(0, GROUPS, fix_body, 0)

        pltpu.sync_copy(acc_v, out_hbm.at[pl.ds(base, BPW)])

    return sc_kernel


_sc_kernel = None


def kernel(item_table, indices, rew, live_mat):
    global _sc_kernel
    if _sc_kernel is None:
        _sc_kernel = _build()
    pairs = item_table[:N_ITEMS].reshape(N_PAIRS, 2 * DIM)
    last = jnp.concatenate([item_table[N_ITEMS:], item_table[N_ITEMS:]],
                           axis=1)
    idx2 = indices.reshape(W, B)
    rew2 = rew.reshape(W, B)
    live_f = live_mat.astype(jnp.float32)
    return _sc_kernel(pairs, last, idx2, rew2, live_f)
